# baseline (speedup 1.0000x reference)
import functools

import jax
import jax.numpy as jnp
import numpy as np
from jax import lax
from jax.experimental import pallas as pl
from jax.experimental.pallas import tpu as pltpu
from jax.experimental.pallas import tpu_sc as plsc

F32 = jnp.float32
I32 = jnp.int32

_XMIN, _XMAX = -3.84, 3.84
_ZMIN = -0.2
_NXG = 77
_NZG = 29
_NAG = 13
_EPS = 1e-4
_XHI = float(np.float32(_NXG - 1 - _EPS))
_ZHI = float(np.float32(_NZG - 1 - _EPS))
_AHI = float(np.float32(_NAG - 1 - _EPS))
_INV_VS = 10.0

_N = 20000
_NP = 20224
_NB = 20480
_KB = 64
_PLANE = _NXG * _NZG
_SLABA = 87168
_TRI_CH = 2528
_BILF = 1008
_LINF = 80
_CB = 160
_NCH = 8


def _cues(bbox):
    c = bbox[:, 0:3]
    l = bbox[:, 3]; w = bbox[:, 4]; h = bbox[:, 5]; th = bbox[:, 6]
    ct = jnp.cos(th); st = jnp.sin(th)
    sx = jnp.array([1, 1, 1, 1, -1, -1, -1, -1], F32)
    sy = jnp.array([1, 1, -1, -1, 1, 1, -1, -1], F32)
    sz = jnp.array([1, -1, 1, -1, 1, -1, 1, -1], F32)
    ox = sx[None, :] * (l / 2)[:, None] * ct[:, None] - sy[None, :] * (w / 2)[:, None] * st[:, None]
    oy = sx[None, :] * (l / 2)[:, None] * st[:, None] + sy[None, :] * (w / 2)[:, None] * ct[:, None]
    oz = sz[None, :] * (h / 2)[:, None]
    corners = c[:, None, :] + jnp.stack([ox, oy, oz], axis=2)
    ang = jnp.mod(th, jnp.pi) / (jnp.pi / 12.0)
    dx = c[:, 0] * ct + c[:, 1] * st
    dy = -c[:, 0] * st + c[:, 1] * ct
    clip = lambda v: jnp.clip(v, _XMIN, _XMAX)
    return (c, corners.reshape(-1, 3), ang,
            clip(dx - l / 2), clip(dx + l / 2), clip(dy - w / 2), clip(dy + w / 2),
            c[:, 2] - h / 2, c[:, 2] + h / 2)


def _pad_to(a, n):
    return jnp.concatenate([a, jnp.zeros((n - a.shape[0],), a.dtype)])


def _sc_body(votes, xang1, yang1, scar, cenp, corp, bilp, linp, vox1, vox2,
             out, slab, vtb, xac, yac, scc, cenb, corb, bilb, linb,
             fbil, flin, vidx, vwb, vgb, acc, sem, vsem, v1sem):
    cidx = lax.axis_index("c")
    sidx = lax.axis_index("s")
    slab_id = sidx % 2
    grp = sidx // 2
    row = sidx * 2 + cidx
    lo = slab_id * 38
    lane = lax.iota(I32, 16)
    lane3 = lane * 3
    lane12 = lane * 12
    lane_eq = [lane == j for j in range(16)]
    zero16 = jnp.zeros((16,), F32)
    one16 = jnp.ones((16,), F32)

    def _grid3(px, py, pz):
        x = jnp.minimum(jnp.maximum((px - _XMIN) * _INV_VS, 0.0), _XHI)
        y = jnp.minimum(jnp.maximum((py - _XMIN) * _INV_VS, 0.0), _XHI)
        z = jnp.minimum(jnp.maximum((pz - _ZMIN) * _INV_VS, 0.0), _ZHI)
        x0 = x.astype(I32); y0 = y.astype(I32); z0 = z.astype(I32)
        return x0, y0, z0, x - x0.astype(F32), y - y0.astype(F32), z - z0.astype(F32)

    def _tri_w(fx, fy, fz, cf):
        ax0 = (1.0 - fx) * cf; ax1 = fx * cf
        gy = 1.0 - fy; gz = 1.0 - fz
        w00 = ax0 * gy; w01 = ax0 * fy; w10 = ax1 * gy; w11 = ax1 * fy
        return (w00 * gz, w00 * fz, w01 * gz, w01 * fz,
                w10 * gz, w10 * fz, w11 * gz, w11 * fz)

    descs = [
        pltpu.async_copy(votes.at[pl.ds((cidx * _NP + grp * _TRI_CH) * 3, _TRI_CH * 3)],
                         vtb, sem),
        pltpu.async_copy(cenp, cenb, sem),
        pltpu.async_copy(corp, corb, sem),
        pltpu.async_copy(bilp, bilb, sem),
        pltpu.async_copy(linp, linb, sem),
    ]

    def _zb(i, _):
        for t in range(8):
            slab[pl.ds(i * 128 + t * 16, 16)] = zero16
        return 0
    lax.fori_loop(0, _SLABA // 128, _zb, 0)

    @pl.when(cidx == 0)
    def _():
        def _zf(i, _):
            fbil[pl.ds(i * 16, 16)] = zero16
            return 0
        lax.fori_loop(0, 4 * _BILF // 16, _zf, 0)

        def _zl(i, _):
            flin[pl.ds(i * 16, 16)] = zero16
            return 0
        lax.fori_loop(0, 2 * _LINF // 16, _zl, 0)
    acc[...] = zero16

    for d in descs:
        d.wait()

    def _vox_prep(pref, npts, gsrc, gdst):
        o = gsrc * 16
        x0, y0, z0, fx, fy, fz = _grid3(pref[pl.ds(o, 16)],
                                        pref[pl.ds(npts + o, 16)],
                                        pref[pl.ds(2 * npts + o, 16)])
        cf = pref[pl.ds(3 * npts + o, 16)]
        b = x0 * _PLANE + y0 * _NZG + z0
        idxs = (b, b + 1, b + _NZG, b + _NZG + 1,
                b + _PLANE, b + _PLANE + 1, b + _PLANE + _NZG, b + _PLANE + _NZG + 1)
        ws = _tri_w(fx, fy, fz, cf)
        for t in range(8):
            vidx[pl.ds(gdst * 128 + t * 16, 16)] = idxs[t]
            vwb[pl.ds(gdst * 128 + t * 16, 16)] = ws[t]

    @pl.when(cidx == 1)
    def _():
        for i in range(4):
            _vox_prep(corb, 1024, sidx * 4 + i, i)
        for i in range(4):
            pltpu.async_copy(vox2.at[vidx.at[pl.ds(i * 128, 128)]],
                             vgb.at[pl.ds(i * 128, 128)], vsem)

    @pl.when((cidx == 1) & (sidx < 8))
    def _():
        _vox_prep(cenb, 128, sidx, 4)
        pltpu.async_copy(vox1.at[vidx.at[pl.ds(512, 128)]],
                         vgb.at[pl.ds(512, 128)], v1sem)

    def _tri_build(pref, npts, ngroups):
        def body(i, _):
            o = i * 16
            x0, y0, z0, fx, fy, fz = _grid3(pref[pl.ds(o, 16)],
                                            pref[pl.ds(npts + o, 16)],
                                            pref[pl.ds(2 * npts + o, 16)])
            cf = pref[pl.ds(3 * npts + o, 16)]
            m0 = (x0 >= lo) & (x0 <= lo + 38)
            m1 = (x0 + 1 >= lo) & (x0 + 1 <= lo + 38)
            yz = y0 * _NZG + z0
            p0 = jnp.minimum(jnp.maximum(x0 - lo, 0), 38)
            p1 = jnp.minimum(jnp.maximum(x0 + 1 - lo, 0), 38)
            b0 = p0 * _PLANE + yz
            b1 = p1 * _PLANE + yz
            idxs = (b0, b0 + 1, b0 + _NZG, b0 + _NZG + 1,
                    b1, b1 + 1, b1 + _NZG, b1 + _NZG + 1)
            ws = _tri_w(fx, fy, fz, cf)
            for j in range(16):
                lm0 = lane_eq[j] & m0
                lm1 = lane_eq[j] & m1
                for t in range(8):
                    plsc.addupdate_scatter(slab, [idxs[t]], ws[t],
                                           mask=lm1 if t >= 4 else lm0)
            return 0
        lax.fori_loop(0, ngroups, body, 0)

    @pl.when(cidx == 0)
    def _():
        _tri_build(cenb, 128, 8)

    @pl.when(cidx == 1)
    def _():
        _tri_build(corb, 1024, 64)

    def _vox_mac(g):
        sacc = zero16
        for t in range(8):
            sacc = sacc + vgb[pl.ds(g * 128 + t * 16, 16)] * vwb[pl.ds(g * 128 + t * 16, 16)]
        acc[...] = acc[...] + sacc

    @pl.when(cidx == 1)
    def _():
        for i in range(4):
            pltpu.make_async_copy(vox2.at[vidx.at[pl.ds(i * 128, 128)]],
                                  vgb.at[pl.ds(i * 128, 128)], vsem).wait()
        for i in range(4):
            _vox_mac(i)

    @pl.when((cidx == 1) & (sidx < 8))
    def _():
        pltpu.make_async_copy(vox1.at[vidx.at[pl.ds(512, 128)]],
                              vgb.at[pl.ds(512, 128)], v1sem).wait()
        _vox_mac(4)

    @pl.when(cidx == 0)
    def _():
        def body(i, _):
            o = i * 16
            a = jnp.minimum(jnp.maximum(bilb[pl.ds(o, 16)], 0.0), _AHI)
            cf = bilb[pl.ds(5 * 128 + o, 16)]
            a0 = a.astype(I32)
            fa = a - a0.astype(F32)
            wa0 = (1.0 - fa) * cf; wa1 = fa * cf
            for f in range(4):
                yv = bilb[pl.ds((1 + f) * 128 + o, 16)]
                y = jnp.minimum(jnp.maximum((yv - _XMIN) * _INV_VS, 0.0), _XHI)
                y0 = y.astype(I32)
                fy = y - y0.astype(F32)
                b = f * _BILF + a0 * _NXG + y0
                idxs = (b, b + 1, b + _NXG, b + _NXG + 1)
                ws = (wa0 * (1.0 - fy), wa0 * fy, wa1 * (1.0 - fy), wa1 * fy)
                for j in range(16):
                    for t in range(4):
                        plsc.addupdate_scatter(fbil, [idxs[t]], ws[t], mask=lane_eq[j])
            cfl = linb[pl.ds(2 * 128 + o, 16)]
            for f in range(2):
                zv = linb[pl.ds(f * 128 + o, 16)]
                z = jnp.minimum(jnp.maximum((zv - _XMIN) * _INV_VS, 0.0), _XHI)
                z0 = z.astype(I32)
                fz = z - z0.astype(F32)
                b = f * _LINF + z0
                w0 = (1.0 - fz) * cfl; w1 = fz * cfl
                for j in range(16):
                    plsc.addupdate_scatter(flin, [b], w0, mask=lane_eq[j])
                    plsc.addupdate_scatter(flin, [b + 1], w1, mask=lane_eq[j])
            return 0
        lax.fori_loop(0, 8, body, 0)

    base_pt = grp * _TRI_CH

    def _tg(i, _):
        o3 = i * 48 + lane3
        xv = plsc.load_gather(vtb, [o3])
        yv = plsc.load_gather(vtb, [o3 + 1])
        zv = plsc.load_gather(vtb, [o3 + 2])
        x0, y0, z0, fx, fy, fz = _grid3(xv, yv, zv)
        pm = (base_pt + i * 16 + lane) < _N
        mm = (x0 >= lo) & (x0 < lo + 38) & pm
        xb = jnp.where(mm, x0, lo)
        b = (xb - lo) * _PLANE + y0 * _NZG + z0
        idxs = (b, b + 1, b + _NZG, b + _NZG + 1,
                b + _PLANE, b + _PLANE + 1, b + _PLANE + _NZG, b + _PLANE + _NZG + 1)
        ws = _tri_w(fx, fy, fz, one16)
        sacc = zero16
        for t in range(8):
            sacc = sacc + plsc.load_gather(slab, [idxs[t]]) * ws[t]
        acc[...] = acc[...] + jnp.where(mm, sacc, 0.0)
        return 0
    lax.fori_loop(0, _TRI_CH // 16, _tg, 0)

    @pl.when(cidx == 0)
    def _():
        tile_p0 = sidx * (_NCH * _CB)

        def fire(c):
            p0 = tile_p0 + c * _CB
            buf = c % 2
            return (
                pltpu.async_copy(xang1.at[pl.ds(p0 * 12, _CB * 12)],
                                 xac.at[pl.ds(buf * _CB * 12, _CB * 12)], sem),
                pltpu.async_copy(yang1.at[pl.ds(p0 * 12, _CB * 12)],
                                 yac.at[pl.ds(buf * _CB * 12, _CB * 12)], sem),
                pltpu.async_copy(scar.at[pl.ds((sidx * _NCH + c) * 8 * _CB, 8 * _CB)],
                                 scc.at[pl.ds(buf * 8 * _CB, 8 * _CB)], sem),
            )

        pend = fire(0)
        for c in range(_NCH):
            for d in pend:
                d.wait()
            if c + 1 < _NCH:
                pend = fire(c + 1)
            buf = c % 2
            abase = buf * _CB * 12
            sbase = buf * 8 * _CB
            cbase = tile_p0 + c * _CB

            def body(i, _):
                o = i * 16
                o12 = abase + i * 192 + lane12

                def argmax12(ref):
                    best = plsc.load_gather(ref, [o12])
                    bidx = zero16
                    for j in range(1, 12):
                        vj = plsc.load_gather(ref, [o12 + j])
                        gtm = vj > best
                        best = jnp.where(gtm, vj, best)
                        bidx = jnp.where(gtm, jnp.full((16,), float(j), F32), bidx)
                    return bidx

                pm = (cbase + o + lane) < _N
                total = zero16
                for (aref, rrow, orow0, f0) in ((xac, 0, 1, 0), (yac, 3, 4, 2)):
                    av = argmax12(aref) + scc[pl.ds(sbase + rrow * _CB + o, 16)]
                    a = jnp.minimum(jnp.maximum(av, 0.0), _AHI)
                    a0 = a.astype(I32)
                    fa = a - a0.astype(F32)
                    for f in range(2):
                        yv = scc[pl.ds(sbase + (orow0 + f) * _CB + o, 16)]
                        y = jnp.minimum(jnp.maximum((yv - _XMIN) * _INV_VS, 0.0), _XHI)
                        y0 = y.astype(I32)
                        fy = y - y0.astype(F32)
                        b = (f0 + f) * _BILF + a0 * _NXG + y0
                        g00 = plsc.load_gather(fbil, [b])
                        g01 = plsc.load_gather(fbil, [b + 1])
                        g10 = plsc.load_gather(fbil, [b + _NXG])
                        g11 = plsc.load_gather(fbil, [b + _NXG + 1])
                        total = total + ((1.0 - fa) * (g00 * (1.0 - fy) + g01 * fy)
                                         + fa * (g10 * (1.0 - fy) + g11 * fy))
                for f in range(2):
                    zv = scc[pl.ds(sbase + (6 + f) * _CB + o, 16)]
                    z = jnp.minimum(jnp.maximum((zv - _XMIN) * _INV_VS, 0.0), _XHI)
                    z0 = z.astype(I32)
                    fz = z - z0.astype(F32)
                    b = f * _LINF + z0
                    total = total + (plsc.load_gather(flin, [b]) * (1.0 - fz)
                                     + plsc.load_gather(flin, [b + 1]) * fz)
                acc[...] = acc[...] + jnp.where(pm, total, 0.0)
                return 0
            lax.fori_loop(0, _CB // 16, body, 0)

    pltpu.sync_copy(acc, out.at[pl.ds(row * 16, 16)])


@functools.partial(
    pl.kernel,
    out_type=jax.ShapeDtypeStruct((512,), F32),
    mesh=plsc.VectorSubcoreMesh(core_axis_name="c", subcore_axis_name="s"),
    compiler_params=pltpu.CompilerParams(needs_layout_passes=False),
    scratch_types=[
        pltpu.VMEM((_SLABA,), F32),
        pltpu.VMEM((3 * _TRI_CH,), F32),
        pltpu.VMEM((2 * 12 * _CB,), F32),
        pltpu.VMEM((2 * 12 * _CB,), F32),
        pltpu.VMEM((2 * 8 * _CB,), F32),
        pltpu.VMEM((4 * 128,), F32),
        pltpu.VMEM((4 * 1024,), F32),
        pltpu.VMEM((6 * 128,), F32),
        pltpu.VMEM((3 * 128,), F32),
        pltpu.VMEM((4 * _BILF,), F32),
        pltpu.VMEM((2 * _LINF,), F32),
        pltpu.VMEM((5 * 128,), I32),
        pltpu.VMEM((5 * 128,), F32),
        pltpu.VMEM((5 * 128,), F32),
        pltpu.VMEM((16,), F32),
        pltpu.SemaphoreType.DMA,
        pltpu.SemaphoreType.DMA,
        pltpu.SemaphoreType.DMA,
    ],
)
def _sc_loss(*refs):
    _sc_body(*refs)


def kernel(vote_xyz_center, vote_xyz_corner, vox_pred1, vox_pred2, z_off0, z_off1,
           x_angle, x_res, x_off0, x_off1, y_angle, y_res, y_off0, y_off1,
           gt_bboxes, pert_bboxes, num_instance):
    padr = lambda a, n: jnp.concatenate([a, jnp.zeros((n - a.shape[0],) + a.shape[1:], F32)])
    votes = jnp.concatenate([padr(vote_xyz_center.reshape(-1, 3), _NP),
                             padr(vote_xyz_corner.reshape(-1, 3), _NP)]).reshape(-1)
    xang1 = padr(x_angle, _NB).reshape(-1)
    yang1 = padr(y_angle, _NB).reshape(-1)
    scar = jnp.stack([_pad_to(x_res, _NB), _pad_to(x_off0, _NB), _pad_to(x_off1, _NB),
                      _pad_to(y_res, _NB), _pad_to(y_off0, _NB), _pad_to(y_off1, _NB),
                      _pad_to(z_off0.reshape(-1), _NB), _pad_to(z_off1.reshape(-1), _NB)])
    scar = scar.reshape(8, 16, 8, _CB).transpose(1, 2, 0, 3).reshape(-1)

    cg = _cues(gt_bboxes.reshape(-1, 7))
    cp = _cues(pert_bboxes.reshape(-1, 7))
    m = (jnp.arange(_KB) < num_instance).astype(F32)
    coef_m = jnp.concatenate([m, -m])
    coef_u = jnp.concatenate([jnp.ones((8 * _KB,), F32), -jnp.ones((8 * _KB,), F32)])
    cen = jnp.concatenate([cg[0], cp[0]], axis=0)
    cor = jnp.concatenate([cg[1], cp[1]], axis=0)
    cenp = jnp.concatenate([cen.T.reshape(-1), coef_m])
    corp = jnp.concatenate([cor.T.reshape(-1), coef_u])
    ang = jnp.concatenate([cg[2], cp[2]])
    bilp = jnp.concatenate([ang,
                            jnp.concatenate([cg[3], cp[3]]), jnp.concatenate([cg[4], cp[4]]),
                            jnp.concatenate([cg[5], cp[5]]), jnp.concatenate([cg[6], cp[6]]),
                            coef_m])
    linp = jnp.concatenate([jnp.concatenate([cg[7], cp[7]]),
                            jnp.concatenate([cg[8], cp[8]]), coef_m])

    vox1 = vox_pred1.reshape(-1)
    vox2 = vox_pred2.reshape(-1)

    out = _sc_loss(votes, xang1, yang1, scar, cenp, corp, bilp, linp, vox1, vox2)
    return jnp.sum(out)


# speedup vs baseline: 2.0323x; 2.0323x over previous
import functools

import jax
import jax.numpy as jnp
import numpy as np
from jax import lax
from jax.experimental import pallas as pl
from jax.experimental.pallas import tpu as pltpu
from jax.experimental.pallas import tpu_sc as plsc

F32 = jnp.float32
I32 = jnp.int32

_XMIN, _XMAX = -3.84, 3.84
_ZMIN = -0.2
_NXG = 77
_NZG = 29
_NAG = 13
_EPS = 1e-4
_XHI = float(np.float32(_NXG - 1 - _EPS))
_ZHI = float(np.float32(_NZG - 1 - _EPS))
_AHI = float(np.float32(_NAG - 1 - _EPS))
_INV_VS = 10.0

_N = 20000
_KB = 64
_PLANE = _NXG * _NZG
_SLABA = 87168
_TRI_OWN = 2500
_TRI_CH = 2512
_BIL_OWN = 1250
_BIL_W = 1264
_CB = 160
_NCH = 8
_BILF = 1008
_LINF = 80


def _cues(bbox):
    c = bbox[:, 0:3]
    l = bbox[:, 3]; w = bbox[:, 4]; h = bbox[:, 5]; th = bbox[:, 6]
    ct = jnp.cos(th); st = jnp.sin(th)
    sx = jnp.array([1, 1, 1, 1, -1, -1, -1, -1], F32)
    sy = jnp.array([1, 1, -1, -1, 1, 1, -1, -1], F32)
    sz = jnp.array([1, -1, 1, -1, 1, -1, 1, -1], F32)
    ox = sx[None, :] * (l / 2)[:, None] * ct[:, None] - sy[None, :] * (w / 2)[:, None] * st[:, None]
    oy = sx[None, :] * (l / 2)[:, None] * st[:, None] + sy[None, :] * (w / 2)[:, None] * ct[:, None]
    oz = sz[None, :] * (h / 2)[:, None]
    corners = c[:, None, :] + jnp.stack([ox, oy, oz], axis=2)
    ang = jnp.mod(th, jnp.pi) / (jnp.pi / 12.0)
    dx = c[:, 0] * ct + c[:, 1] * st
    dy = -c[:, 0] * st + c[:, 1] * ct
    clip = lambda v: jnp.clip(v, _XMIN, _XMAX)
    return (c, corners.reshape(-1, 3), ang,
            clip(dx - l / 2), clip(dx + l / 2), clip(dy - w / 2), clip(dy + w / 2),
            c[:, 2] - h / 2, c[:, 2] + h / 2)


def _sc_body(vcx, vcy, vcz, vkx, vky, vkz,
             xaf, yaf, xr, xo0, xo1, yr, yo0, yo1, zf0, zf1,
             cenp, corp, bilp, linp, vox1, vox2,
             out, slab, vtb, sbuf, cenb, corb, bilb, linb,
             fbil, flin, vidx, vwb, vgb, acc, sem, vsem, v1sem):
    cidx = lax.axis_index("c")
    sidx = lax.axis_index("s")
    slab_id = sidx % 2
    grp = sidx // 2
    row = sidx * 2 + cidx
    lo = slab_id * 38
    lane = lax.iota(I32, 16)
    lane_eq = [lane == j for j in range(16)]
    zero16 = jnp.zeros((16,), F32)
    one16 = jnp.ones((16,), F32)

    def _grid3(px, py, pz):
        x = jnp.minimum(jnp.maximum((px - _XMIN) * _INV_VS, 0.0), _XHI)
        y = jnp.minimum(jnp.maximum((py - _XMIN) * _INV_VS, 0.0), _XHI)
        z = jnp.minimum(jnp.maximum((pz - _ZMIN) * _INV_VS, 0.0), _ZHI)
        x0 = x.astype(I32); y0 = y.astype(I32); z0 = z.astype(I32)
        return x0, y0, z0, x - x0.astype(F32), y - y0.astype(F32), z - z0.astype(F32)

    def _tri_w(fx, fy, fz, cf):
        ax0 = (1.0 - fx) * cf; ax1 = fx * cf
        gy = 1.0 - fy; gz = 1.0 - fz
        w00 = ax0 * gy; w01 = ax0 * fy; w10 = ax1 * gy; w11 = ax1 * fy
        return (w00 * gz, w00 * fz, w01 * gz, w01 * fz,
                w10 * gz, w10 * fz, w11 * gz, w11 * fz)

    tri_a = pl.multiple_of(grp * _TRI_OWN - (grp * 4) % 16, 16)

    @pl.when(cidx == 0)
    def _():
        for j, r in enumerate((vcx, vcy, vcz)):
            pltpu.async_copy(r.at[pl.ds(tri_a, _TRI_CH)],
                             vtb.at[pl.ds(j * _TRI_CH, _TRI_CH)], sem)

    @pl.when(cidx == 1)
    def _():
        for j, r in enumerate((vkx, vky, vkz)):
            pltpu.async_copy(r.at[pl.ds(tri_a, _TRI_CH)],
                             vtb.at[pl.ds(j * _TRI_CH, _TRI_CH)], sem)

    descs = [
        pltpu.async_copy(cenp, cenb, sem),
        pltpu.async_copy(corp, corb, sem),
        pltpu.async_copy(bilp, bilb, sem),
        pltpu.async_copy(linp, linb, sem),
    ]

    def _zb(i, _):
        for t in range(8):
            slab[pl.ds(i * 128 + t * 16, 16)] = zero16
        return 0
    lax.fori_loop(0, _SLABA // 128, _zb, 0)

    @pl.when(cidx == 0)
    def _():
        def _zf(i, _):
            fbil[pl.ds(i * 16, 16)] = zero16
            return 0
        lax.fori_loop(0, 4 * _BILF // 16, _zf, 0)

        def _zl(i, _):
            flin[pl.ds(i * 16, 16)] = zero16
            return 0
        lax.fori_loop(0, 2 * _LINF // 16, _zl, 0)
    acc[...] = zero16

    for d in descs:
        d.wait()
    for j in range(3):
        pltpu.make_async_copy(vcx.at[pl.ds(tri_a, _TRI_CH)],
                              vtb.at[pl.ds(j * _TRI_CH, _TRI_CH)], sem).wait()

    def _vox_prep(pref, npts, gsrc, gdst):
        o = gsrc * 16
        x0, y0, z0, fx, fy, fz = _grid3(pref[pl.ds(o, 16)],
                                        pref[pl.ds(npts + o, 16)],
                                        pref[pl.ds(2 * npts + o, 16)])
        cf = pref[pl.ds(3 * npts + o, 16)]
        b = x0 * _PLANE + y0 * _NZG + z0
        idxs = (b, b + 1, b + _NZG, b + _NZG + 1,
                b + _PLANE, b + _PLANE + 1, b + _PLANE + _NZG, b + _PLANE + _NZG + 1)
        ws = _tri_w(fx, fy, fz, cf)
        for t in range(8):
            vidx[pl.ds(gdst * 128 + t * 16, 16)] = idxs[t]
            vwb[pl.ds(gdst * 128 + t * 16, 16)] = ws[t]

    @pl.when(cidx == 1)
    def _():
        for i in range(4):
            _vox_prep(corb, 1024, sidx * 4 + i, i)
        for i in range(4):
            pltpu.async_copy(vox2.at[vidx.at[pl.ds(i * 128, 128)]],
                             vgb.at[pl.ds(i * 128, 128)], vsem)

    @pl.when((cidx == 1) & (sidx < 8))
    def _():
        _vox_prep(cenb, 128, sidx, 4)
        pltpu.async_copy(vox1.at[vidx.at[pl.ds(512, 128)]],
                         vgb.at[pl.ds(512, 128)], v1sem)

    def _tri_build(pref, npts, ngroups):
        def body(i, _):
            o = i * 16
            x0, y0, z0, fx, fy, fz = _grid3(pref[pl.ds(o, 16)],
                                            pref[pl.ds(npts + o, 16)],
                                            pref[pl.ds(2 * npts + o, 16)])
            cf = pref[pl.ds(3 * npts + o, 16)]
            m0 = (x0 >= lo) & (x0 <= lo + 38)
            m1 = (x0 + 1 >= lo) & (x0 + 1 <= lo + 38)
            yz = y0 * _NZG + z0
            p0 = jnp.minimum(jnp.maximum(x0 - lo, 0), 38)
            p1 = jnp.minimum(jnp.maximum(x0 + 1 - lo, 0), 38)
            b0 = p0 * _PLANE + yz
            b1 = p1 * _PLANE + yz
            idxs = (b0, b0 + 1, b0 + _NZG, b0 + _NZG + 1,
                    b1, b1 + 1, b1 + _NZG, b1 + _NZG + 1)
            ws = _tri_w(fx, fy, fz, cf)
            for j in range(16):
                lm0 = lane_eq[j] & m0
                lm1 = lane_eq[j] & m1
                for t in range(8):
                    plsc.addupdate_scatter(slab, [idxs[t]], ws[t],
                                           mask=lm1 if t >= 4 else lm0)
            return 0
        lax.fori_loop(0, ngroups, body, 0)

    @pl.when(cidx == 0)
    def _():
        _tri_build(cenb, 128, 8)

    @pl.when(cidx == 1)
    def _():
        _tri_build(corb, 1024, 64)

    def _vox_mac(g):
        sacc = zero16
        for t in range(8):
            sacc = sacc + vgb[pl.ds(g * 128 + t * 16, 16)] * vwb[pl.ds(g * 128 + t * 16, 16)]
        acc[...] = acc[...] + sacc

    @pl.when(cidx == 1)
    def _():
        for i in range(4):
            pltpu.make_async_copy(vox2.at[vidx.at[pl.ds(i * 128, 128)]],
                                  vgb.at[pl.ds(i * 128, 128)], vsem).wait()
        for i in range(4):
            _vox_mac(i)

    @pl.when((cidx == 1) & (sidx < 8))
    def _():
        pltpu.make_async_copy(vox1.at[vidx.at[pl.ds(512, 128)]],
                              vgb.at[pl.ds(512, 128)], v1sem).wait()
        _vox_mac(4)

    @pl.when(cidx == 0)
    def _():
        def body(i, _):
            o = i * 16
            a = jnp.minimum(jnp.maximum(bilb[pl.ds(o, 16)], 0.0), _AHI)
            cf = bilb[pl.ds(5 * 128 + o, 16)]
            a0 = a.astype(I32)
            fa = a - a0.astype(F32)
            wa0 = (1.0 - fa) * cf; wa1 = fa * cf
            for f in range(4):
                yv = bilb[pl.ds((1 + f) * 128 + o, 16)]
                y = jnp.minimum(jnp.maximum((yv - _XMIN) * _INV_VS, 0.0), _XHI)
                y0 = y.astype(I32)
                fy = y - y0.astype(F32)
                b = f * _BILF + a0 * _NXG + y0
                idxs = (b, b + 1, b + _NXG, b + _NXG + 1)
                ws = (wa0 * (1.0 - fy), wa0 * fy, wa1 * (1.0 - fy), wa1 * fy)
                for j in range(16):
                    for t in range(4):
                        plsc.addupdate_scatter(fbil, [idxs[t]], ws[t], mask=lane_eq[j])
            cfl = linb[pl.ds(2 * 128 + o, 16)]
            for f in range(2):
                zv = linb[pl.ds(f * 128 + o, 16)]
                z = jnp.minimum(jnp.maximum((zv - _XMIN) * _INV_VS, 0.0), _XHI)
                z0 = z.astype(I32)
                fz = z - z0.astype(F32)
                b = f * _LINF + z0
                w0 = (1.0 - fz) * cfl; w1 = fz * cfl
                for j in range(16):
                    plsc.addupdate_scatter(flin, [b], w0, mask=lane_eq[j])
                    plsc.addupdate_scatter(flin, [b + 1], w1, mask=lane_eq[j])
            return 0
        lax.fori_loop(0, 8, body, 0)

    glo = grp * _TRI_OWN

    def _tg(i, _):
        o = i * 16
        x0, y0, z0, fx, fy, fz = _grid3(vtb[pl.ds(o, 16)],
                                        vtb[pl.ds(_TRI_CH + o, 16)],
                                        vtb[pl.ds(2 * _TRI_CH + o, 16)])
        pos = tri_a + o + lane
        pm = (pos >= glo) & (pos < glo + _TRI_OWN)
        mm = (x0 >= lo) & (x0 < lo + 38) & pm
        xb = jnp.where(mm, x0, lo)
        b = (xb - lo) * _PLANE + y0 * _NZG + z0
        idxs = (b, b + 1, b + _NZG, b + _NZG + 1,
                b + _PLANE, b + _PLANE + 1, b + _PLANE + _NZG, b + _PLANE + _NZG + 1)
        ws = _tri_w(fx, fy, fz, one16)
        sacc = zero16
        for t in range(8):
            sacc = sacc + plsc.load_gather(slab, [idxs[t]]) * ws[t]
        acc[...] = acc[...] + jnp.where(mm, sacc, 0.0)
        return 0
    lax.fori_loop(0, _TRI_CH // 16, _tg, 0)

    srcs = (xaf, yaf, xr, xo0, xo1, yr, yo0, yo1, zf0, zf1)

    @pl.when(cidx == 0)
    def _():
        bil_a = pl.multiple_of(sidx * _BIL_OWN - (sidx * 2) % 16, 16)
        blo = sidx * _BIL_OWN

        def fire(c):
            csz = _CB if c + 1 < _NCH else (_BIL_W - (_NCH - 1) * _CB)
            buf = c % 2
            return [pltpu.async_copy(
                        r.at[pl.ds(bil_a + c * _CB, csz)],
                        sbuf.at[pl.ds((buf * 10 + j) * _CB, csz)], sem)
                    for j, r in enumerate(srcs)]

        pend = fire(0)
        for c in range(_NCH):
            csz = _CB if c + 1 < _NCH else (_BIL_W - (_NCH - 1) * _CB)
            for d in pend:
                d.wait()
            if c + 1 < _NCH:
                pend = fire(c + 1)
            sb = (c % 2) * 10 * _CB

            def body(i, _):
                o = i * 16

                def rrow(j):
                    return sbuf[pl.ds(sb + j * _CB + o, 16)]

                pos = bil_a + c * _CB + o + lane
                pm = (pos >= blo) & (pos < blo + _BIL_OWN)
                total = zero16
                for (arow, rres, orow0, f0) in ((0, 2, 3, 0), (1, 5, 6, 2)):
                    av = rrow(arow) + rrow(rres)
                    a = jnp.minimum(jnp.maximum(av, 0.0), _AHI)
                    a0 = a.astype(I32)
                    fa = a - a0.astype(F32)
                    for f in range(2):
                        yv = rrow(orow0 + f)
                        y = jnp.minimum(jnp.maximum((yv - _XMIN) * _INV_VS, 0.0), _XHI)
                        y0 = y.astype(I32)
                        fy = y - y0.astype(F32)
                        b = (f0 + f) * _BILF + a0 * _NXG + y0
                        g00 = plsc.load_gather(fbil, [b])
                        g01 = plsc.load_gather(fbil, [b + 1])
                        g10 = plsc.load_gather(fbil, [b + _NXG])
                        g11 = plsc.load_gather(fbil, [b + _NXG + 1])
                        total = total + ((1.0 - fa) * (g00 * (1.0 - fy) + g01 * fy)
                                         + fa * (g10 * (1.0 - fy) + g11 * fy))
                for f in range(2):
                    zv = rrow(8 + f)
                    z = jnp.minimum(jnp.maximum((zv - _XMIN) * _INV_VS, 0.0), _XHI)
                    z0 = z.astype(I32)
                    fz = z - z0.astype(F32)
                    b = f * _LINF + z0
                    total = total + (plsc.load_gather(flin, [b]) * (1.0 - fz)
                                     + plsc.load_gather(flin, [b + 1]) * fz)
                acc[...] = acc[...] + jnp.where(pm, total, 0.0)
                return 0
            lax.fori_loop(0, csz // 16, body, 0)

    pltpu.sync_copy(acc, out.at[pl.ds(row * 16, 16)])


@functools.partial(
    pl.kernel,
    out_type=jax.ShapeDtypeStruct((512,), F32),
    mesh=plsc.VectorSubcoreMesh(core_axis_name="c", subcore_axis_name="s"),
    compiler_params=pltpu.CompilerParams(needs_layout_passes=False),
    scratch_types=[
        pltpu.VMEM((_SLABA,), F32),
        pltpu.VMEM((3 * _TRI_CH,), F32),
        pltpu.VMEM((2 * 10 * _CB,), F32),
        pltpu.VMEM((4 * 128,), F32),
        pltpu.VMEM((4 * 1024,), F32),
        pltpu.VMEM((6 * 128,), F32),
        pltpu.VMEM((3 * 128,), F32),
        pltpu.VMEM((4 * _BILF,), F32),
        pltpu.VMEM((2 * _LINF,), F32),
        pltpu.VMEM((5 * 128,), I32),
        pltpu.VMEM((5 * 128,), F32),
        pltpu.VMEM((5 * 128,), F32),
        pltpu.VMEM((16,), F32),
        pltpu.SemaphoreType.DMA,
        pltpu.SemaphoreType.DMA,
        pltpu.SemaphoreType.DMA,
    ],
)
def _sc_loss(*refs):
    _sc_body(*refs)


def kernel(vote_xyz_center, vote_xyz_corner, vox_pred1, vox_pred2, z_off0, z_off1,
           x_angle, x_res, x_off0, x_off1, y_angle, y_res, y_off0, y_off1,
           gt_bboxes, pert_bboxes, num_instance):
    vcx = vote_xyz_center[0, :, 0]; vcy = vote_xyz_center[0, :, 1]; vcz = vote_xyz_center[0, :, 2]
    vkx = vote_xyz_corner[0, :, 0]; vky = vote_xyz_corner[0, :, 1]; vkz = vote_xyz_corner[0, :, 2]
    xaf = jnp.argmax(x_angle, axis=1).astype(F32)
    yaf = jnp.argmax(y_angle, axis=1).astype(F32)
    zf0 = z_off0.reshape(-1)
    zf1 = z_off1.reshape(-1)

    cg = _cues(gt_bboxes.reshape(-1, 7))
    cp = _cues(pert_bboxes.reshape(-1, 7))
    m = (jnp.arange(_KB) < num_instance).astype(F32)
    coef_m = jnp.concatenate([m, -m])
    coef_u = jnp.concatenate([jnp.ones((8 * _KB,), F32), -jnp.ones((8 * _KB,), F32)])
    cen = jnp.concatenate([cg[0], cp[0]], axis=0)
    cor = jnp.concatenate([cg[1], cp[1]], axis=0)
    cenp = jnp.concatenate([cen.T.reshape(-1), coef_m])
    corp = jnp.concatenate([cor.T.reshape(-1), coef_u])
    ang = jnp.concatenate([cg[2], cp[2]])
    bilp = jnp.concatenate([ang,
                            jnp.concatenate([cg[3], cp[3]]), jnp.concatenate([cg[4], cp[4]]),
                            jnp.concatenate([cg[5], cp[5]]), jnp.concatenate([cg[6], cp[6]]),
                            coef_m])
    linp = jnp.concatenate([jnp.concatenate([cg[7], cp[7]]),
                            jnp.concatenate([cg[8], cp[8]]), coef_m])

    vox1 = vox_pred1.reshape(-1)
    vox2 = vox_pred2.reshape(-1)

    out = _sc_loss(vcx, vcy, vcz, vkx, vky, vkz,
                   xaf, yaf, x_res, x_off0, x_off1, y_res, y_off0, y_off1, zf0, zf1,
                   cenp, corp, bilp, linp, vox1, vox2)
    return jnp.sum(out)


# speedup vs baseline: 2.2072x; 1.0860x over previous
import functools

import jax
import jax.numpy as jnp
import numpy as np
from jax import lax
from jax.experimental import pallas as pl
from jax.experimental.pallas import tpu as pltpu
from jax.experimental.pallas import tpu_sc as plsc

F32 = jnp.float32
I32 = jnp.int32

_XMIN, _XMAX = -3.84, 3.84
_ZMIN = -0.2
_NXG = 77
_NZG = 29
_NAG = 13
_EPS = 1e-4
_XHI = float(np.float32(_NXG - 1 - _EPS))
_ZHI = float(np.float32(_NZG - 1 - _EPS))
_AHI = float(np.float32(_NAG - 1 - _EPS))
_INV_VS = 10.0

_N = 20000
_KB = 64
_PLANE = _NXG * _NZG
_SLABA = 87168
_TRI_OWN = 2500
_TRI_CH = 2512
_BIL_OWN = 1250
_BIL_W = 1264
_CB = 160
_NCH = 8
_BILF = 1008
_LINF = 80


def _cues(bbox):
    c = bbox[:, 0:3]
    l = bbox[:, 3]; w = bbox[:, 4]; h = bbox[:, 5]; th = bbox[:, 6]
    ct = jnp.cos(th); st = jnp.sin(th)
    sx = jnp.array([1, 1, 1, 1, -1, -1, -1, -1], F32)
    sy = jnp.array([1, 1, -1, -1, 1, 1, -1, -1], F32)
    sz = jnp.array([1, -1, 1, -1, 1, -1, 1, -1], F32)
    ox = sx[None, :] * (l / 2)[:, None] * ct[:, None] - sy[None, :] * (w / 2)[:, None] * st[:, None]
    oy = sx[None, :] * (l / 2)[:, None] * st[:, None] + sy[None, :] * (w / 2)[:, None] * ct[:, None]
    oz = sz[None, :] * (h / 2)[:, None]
    corners = c[:, None, :] + jnp.stack([ox, oy, oz], axis=2)
    ang = jnp.mod(th, jnp.pi) / (jnp.pi / 12.0)
    dx = c[:, 0] * ct + c[:, 1] * st
    dy = -c[:, 0] * st + c[:, 1] * ct
    clip = lambda v: jnp.clip(v, _XMIN, _XMAX)
    return (c, corners.reshape(-1, 3), ang,
            clip(dx - l / 2), clip(dx + l / 2), clip(dy - w / 2), clip(dy + w / 2),
            c[:, 2] - h / 2, c[:, 2] + h / 2)


def _sc_body(vcx, vcy, vcz, vkx, vky, vkz,
             axf, ayf, xo0, xo1, yo0, yo1, zf0, zf1,
             cuep, vox1, vox2,
             out, slab, vtb, sbuf, cueb,
             fbil, flin, vidx, vwb, vgb, acc, sem, vsem, v1sem):
    cidx = lax.axis_index("c")
    sidx = lax.axis_index("s")
    slab_id = sidx % 2
    grp = sidx // 2
    row = sidx * 2 + cidx
    lo = slab_id * 38
    lane = lax.iota(I32, 16)
    lane_eq = [lane == j for j in range(16)]
    zero16 = jnp.zeros((16,), F32)
    one16 = jnp.ones((16,), F32)

    def _grid3(px, py, pz):
        x = jnp.minimum(jnp.maximum((px - _XMIN) * _INV_VS, 0.0), _XHI)
        y = jnp.minimum(jnp.maximum((py - _XMIN) * _INV_VS, 0.0), _XHI)
        z = jnp.minimum(jnp.maximum((pz - _ZMIN) * _INV_VS, 0.0), _ZHI)
        x0 = x.astype(I32); y0 = y.astype(I32); z0 = z.astype(I32)
        return x0, y0, z0, x - x0.astype(F32), y - y0.astype(F32), z - z0.astype(F32)

    def _tri_w(fx, fy, fz, cf):
        ax0 = (1.0 - fx) * cf; ax1 = fx * cf
        gy = 1.0 - fy; gz = 1.0 - fz
        w00 = ax0 * gy; w01 = ax0 * fy; w10 = ax1 * gy; w11 = ax1 * fy
        return (w00 * gz, w00 * fz, w01 * gz, w01 * fz,
                w10 * gz, w10 * fz, w11 * gz, w11 * fz)

    tri_a = pl.multiple_of(grp * _TRI_OWN - (grp * 4) % 16, 16)

    @pl.when(cidx == 0)
    def _():
        for j, r in enumerate((vcx, vcy, vcz)):
            pltpu.async_copy(r.at[pl.ds(tri_a, _TRI_CH)],
                             vtb.at[pl.ds(j * _TRI_CH, _TRI_CH)], sem)

    @pl.when(cidx == 1)
    def _():
        for j, r in enumerate((vkx, vky, vkz)):
            pltpu.async_copy(r.at[pl.ds(tri_a, _TRI_CH)],
                             vtb.at[pl.ds(j * _TRI_CH, _TRI_CH)], sem)

    descs = [pltpu.async_copy(cuep, cueb, sem)]

    def _zb(i, _):
        for t in range(8):
            slab[pl.ds(i * 128 + t * 16, 16)] = zero16
        return 0
    lax.fori_loop(0, _SLABA // 128, _zb, 0)

    @pl.when(cidx == 0)
    def _():
        def _zf(i, _):
            fbil[pl.ds(i * 16, 16)] = zero16
            return 0
        lax.fori_loop(0, 4 * _BILF // 16, _zf, 0)

        def _zl(i, _):
            flin[pl.ds(i * 16, 16)] = zero16
            return 0
        lax.fori_loop(0, 2 * _LINF // 16, _zl, 0)
    acc[...] = zero16

    for d in descs:
        d.wait()
    for j in range(3):
        pltpu.make_async_copy(vcx.at[pl.ds(tri_a, _TRI_CH)],
                              vtb.at[pl.ds(j * _TRI_CH, _TRI_CH)], sem).wait()

    def _vox_prep(base, npts, gsrc, gdst):
        o = base + gsrc * 16
        x0, y0, z0, fx, fy, fz = _grid3(cueb[pl.ds(o, 16)],
                                        cueb[pl.ds(npts + o, 16)],
                                        cueb[pl.ds(2 * npts + o, 16)])
        cf = cueb[pl.ds(3 * npts + o, 16)]
        b = x0 * _PLANE + y0 * _NZG + z0
        idxs = (b, b + 1, b + _NZG, b + _NZG + 1,
                b + _PLANE, b + _PLANE + 1, b + _PLANE + _NZG, b + _PLANE + _NZG + 1)
        ws = _tri_w(fx, fy, fz, cf)
        for t in range(8):
            vidx[pl.ds(gdst * 128 + t * 16, 16)] = idxs[t]
            vwb[pl.ds(gdst * 128 + t * 16, 16)] = ws[t]

    @pl.when(cidx == 1)
    def _():
        for i in range(4):
            _vox_prep(512, 1024, sidx * 4 + i, i)
        for i in range(4):
            pltpu.async_copy(vox2.at[vidx.at[pl.ds(i * 128, 128)]],
                             vgb.at[pl.ds(i * 128, 128)], vsem)

    @pl.when((cidx == 1) & (sidx < 8))
    def _():
        _vox_prep(0, 128, sidx, 4)
        pltpu.async_copy(vox1.at[vidx.at[pl.ds(512, 128)]],
                         vgb.at[pl.ds(512, 128)], v1sem)

    def _tri_build(base, npts, ngroups):
        def body(i, _):
            o = base + i * 16
            x0, y0, z0, fx, fy, fz = _grid3(cueb[pl.ds(o, 16)],
                                            cueb[pl.ds(npts + o, 16)],
                                            cueb[pl.ds(2 * npts + o, 16)])
            cf = cueb[pl.ds(3 * npts + o, 16)]
            m0 = (x0 >= lo) & (x0 <= lo + 38)
            m1 = (x0 + 1 >= lo) & (x0 + 1 <= lo + 38)
            yz = y0 * _NZG + z0
            p0 = jnp.minimum(jnp.maximum(x0 - lo, 0), 38)
            p1 = jnp.minimum(jnp.maximum(x0 + 1 - lo, 0), 38)
            b0 = p0 * _PLANE + yz
            b1 = p1 * _PLANE + yz
            idxs = (b0, b0 + 1, b0 + _NZG, b0 + _NZG + 1,
                    b1, b1 + 1, b1 + _NZG, b1 + _NZG + 1)
            ws = _tri_w(fx, fy, fz, cf)
            for j in range(16):
                lm0 = lane_eq[j] & m0
                lm1 = lane_eq[j] & m1
                for t in range(8):
                    plsc.addupdate_scatter(slab, [idxs[t]], ws[t],
                                           mask=lm1 if t >= 4 else lm0)
            return 0
        lax.fori_loop(0, ngroups, body, 0)

    @pl.when(cidx == 0)
    def _():
        _tri_build(0, 128, 8)

    @pl.when(cidx == 1)
    def _():
        _tri_build(512, 1024, 64)

    def _vox_mac(g):
        sacc = zero16
        for t in range(8):
            sacc = sacc + vgb[pl.ds(g * 128 + t * 16, 16)] * vwb[pl.ds(g * 128 + t * 16, 16)]
        acc[...] = acc[...] + sacc

    @pl.when(cidx == 1)
    def _():
        for i in range(4):
            pltpu.make_async_copy(vox2.at[vidx.at[pl.ds(i * 128, 128)]],
                                  vgb.at[pl.ds(i * 128, 128)], vsem).wait()
        for i in range(4):
            _vox_mac(i)

    @pl.when((cidx == 1) & (sidx < 8))
    def _():
        pltpu.make_async_copy(vox1.at[vidx.at[pl.ds(512, 128)]],
                              vgb.at[pl.ds(512, 128)], v1sem).wait()
        _vox_mac(4)

    @pl.when(cidx == 0)
    def _():
        def body(i, _):
            o = i * 16
            a = jnp.minimum(jnp.maximum(cueb[pl.ds(4608 + o, 16)], 0.0), _AHI)
            cf = cueb[pl.ds(4608 + 5 * 128 + o, 16)]
            a0 = a.astype(I32)
            fa = a - a0.astype(F32)
            wa0 = (1.0 - fa) * cf; wa1 = fa * cf
            for f in range(4):
                yv = cueb[pl.ds(4608 + (1 + f) * 128 + o, 16)]
                y = jnp.minimum(jnp.maximum((yv - _XMIN) * _INV_VS, 0.0), _XHI)
                y0 = y.astype(I32)
                fy = y - y0.astype(F32)
                b = f * _BILF + a0 * _NXG + y0
                idxs = (b, b + 1, b + _NXG, b + _NXG + 1)
                ws = (wa0 * (1.0 - fy), wa0 * fy, wa1 * (1.0 - fy), wa1 * fy)
                for j in range(16):
                    for t in range(4):
                        plsc.addupdate_scatter(fbil, [idxs[t]], ws[t], mask=lane_eq[j])
            cfl = cueb[pl.ds(5376 + 2 * 128 + o, 16)]
            for f in range(2):
                zv = cueb[pl.ds(5376 + f * 128 + o, 16)]
                z = jnp.minimum(jnp.maximum((zv - _XMIN) * _INV_VS, 0.0), _XHI)
                z0 = z.astype(I32)
                fz = z - z0.astype(F32)
                b = f * _LINF + z0
                w0 = (1.0 - fz) * cfl; w1 = fz * cfl
                for j in range(16):
                    plsc.addupdate_scatter(flin, [b], w0, mask=lane_eq[j])
                    plsc.addupdate_scatter(flin, [b + 1], w1, mask=lane_eq[j])
            return 0
        lax.fori_loop(0, 8, body, 0)

    glo = grp * _TRI_OWN

    def _tg(i, _):
        o = i * 16
        x0, y0, z0, fx, fy, fz = _grid3(vtb[pl.ds(o, 16)],
                                        vtb[pl.ds(_TRI_CH + o, 16)],
                                        vtb[pl.ds(2 * _TRI_CH + o, 16)])
        pos = tri_a + o + lane
        pm = (pos >= glo) & (pos < glo + _TRI_OWN)
        mm = (x0 >= lo) & (x0 < lo + 38) & pm
        xb = jnp.where(mm, x0, lo)
        b = (xb - lo) * _PLANE + y0 * _NZG + z0
        idxs = (b, b + 1, b + _NZG, b + _NZG + 1,
                b + _PLANE, b + _PLANE + 1, b + _PLANE + _NZG, b + _PLANE + _NZG + 1)
        ws = _tri_w(fx, fy, fz, one16)
        sacc = zero16
        for t in range(8):
            sacc = sacc + plsc.load_gather(slab, [idxs[t]]) * ws[t]
        acc[...] = acc[...] + jnp.where(mm, sacc, 0.0)
        return 0
    lax.fori_loop(0, _TRI_CH // 16, _tg, 0)

    srcs = (axf, ayf, xo0, xo1, yo0, yo1, zf0, zf1)

    @pl.when(cidx == 0)
    def _():
        bil_a = pl.multiple_of(sidx * _BIL_OWN - (sidx * 2) % 16, 16)
        blo = sidx * _BIL_OWN

        def fire(c):
            csz = _CB if c + 1 < _NCH else (_BIL_W - (_NCH - 1) * _CB)
            buf = c % 2
            return [pltpu.async_copy(
                        r.at[pl.ds(bil_a + c * _CB, csz)],
                        sbuf.at[pl.ds((buf * 8 + j) * _CB, csz)], sem)
                    for j, r in enumerate(srcs)]

        pend = fire(0)
        for c in range(_NCH):
            csz = _CB if c + 1 < _NCH else (_BIL_W - (_NCH - 1) * _CB)
            for d in pend:
                d.wait()
            if c + 1 < _NCH:
                pend = fire(c + 1)
            sb = (c % 2) * 8 * _CB

            def body(i, _):
                o = i * 16

                def rrow(j):
                    return sbuf[pl.ds(sb + j * _CB + o, 16)]

                pos = bil_a + c * _CB + o + lane
                pm = (pos >= blo) & (pos < blo + _BIL_OWN)
                total = zero16
                for (arow, orow0, f0) in ((0, 2, 0), (1, 4, 2)):
                    a = rrow(arow)
                    a0 = a.astype(I32)
                    fa = a - a0.astype(F32)
                    for f in range(2):
                        yv = rrow(orow0 + f)
                        y = jnp.minimum(jnp.maximum((yv - _XMIN) * _INV_VS, 0.0), _XHI)
                        y0 = y.astype(I32)
                        fy = y - y0.astype(F32)
                        b = (f0 + f) * _BILF + a0 * _NXG + y0
                        g00 = plsc.load_gather(fbil, [b])
                        g01 = plsc.load_gather(fbil, [b + 1])
                        g10 = plsc.load_gather(fbil, [b + _NXG])
                        g11 = plsc.load_gather(fbil, [b + _NXG + 1])
                        total = total + ((1.0 - fa) * (g00 * (1.0 - fy) + g01 * fy)
                                         + fa * (g10 * (1.0 - fy) + g11 * fy))
                for f in range(2):
                    zv = rrow(6 + f)
                    z = jnp.minimum(jnp.maximum((zv - _XMIN) * _INV_VS, 0.0), _XHI)
                    z0 = z.astype(I32)
                    fz = z - z0.astype(F32)
                    b = f * _LINF + z0
                    total = total + (plsc.load_gather(flin, [b]) * (1.0 - fz)
                                     + plsc.load_gather(flin, [b + 1]) * fz)
                acc[...] = acc[...] + jnp.where(pm, total, 0.0)
                return 0
            lax.fori_loop(0, csz // 16, body, 0)

    pltpu.sync_copy(acc, out.at[pl.ds(row * 16, 16)])


@functools.partial(
    pl.kernel,
    out_type=jax.ShapeDtypeStruct((512,), F32),
    mesh=plsc.VectorSubcoreMesh(core_axis_name="c", subcore_axis_name="s"),
    compiler_params=pltpu.CompilerParams(needs_layout_passes=False),
    scratch_types=[
        pltpu.VMEM((_SLABA,), F32),
        pltpu.VMEM((3 * _TRI_CH,), F32),
        pltpu.VMEM((2 * 8 * _CB,), F32),
        pltpu.VMEM((5760,), F32),
        pltpu.VMEM((4 * _BILF,), F32),
        pltpu.VMEM((2 * _LINF,), F32),
        pltpu.VMEM((5 * 128,), I32),
        pltpu.VMEM((5 * 128,), F32),
        pltpu.VMEM((5 * 128,), F32),
        pltpu.VMEM((16,), F32),
        pltpu.SemaphoreType.DMA,
        pltpu.SemaphoreType.DMA,
        pltpu.SemaphoreType.DMA,
    ],
)
def _sc_loss(*refs):
    _sc_body(*refs)


def kernel(vote_xyz_center, vote_xyz_corner, vox_pred1, vox_pred2, z_off0, z_off1,
           x_angle, x_res, x_off0, x_off1, y_angle, y_res, y_off0, y_off1,
           gt_bboxes, pert_bboxes, num_instance):
    vcx = vote_xyz_center[0, :, 0]; vcy = vote_xyz_center[0, :, 1]; vcz = vote_xyz_center[0, :, 2]
    vkx = vote_xyz_corner[0, :, 0]; vky = vote_xyz_corner[0, :, 1]; vkz = vote_xyz_corner[0, :, 2]
    axf = jnp.clip(jnp.argmax(x_angle, axis=1).astype(F32) + x_res, 0.0, _AHI)
    ayf = jnp.clip(jnp.argmax(y_angle, axis=1).astype(F32) + y_res, 0.0, _AHI)
    zf0 = z_off0.reshape(-1)
    zf1 = z_off1.reshape(-1)

    bb = jnp.concatenate([gt_bboxes.reshape(-1, 7), pert_bboxes.reshape(-1, 7)])
    cu = _cues(bb)
    m = (jnp.arange(_KB) < num_instance).astype(F32)
    coef_m = jnp.concatenate([m, -m])
    coef_u = jnp.concatenate([jnp.ones((8 * _KB,), F32), -jnp.ones((8 * _KB,), F32)])
    cuep = jnp.concatenate([
        cu[0].T.reshape(-1), coef_m,
        cu[1].T.reshape(-1), coef_u,
        cu[2], cu[3], cu[4], cu[5], cu[6], coef_m,
        cu[7], cu[8], coef_m,
    ])

    vox1 = vox_pred1.reshape(-1)
    vox2 = vox_pred2.reshape(-1)

    out = _sc_loss(vcx, vcy, vcz, vkx, vky, vkz,
                   axf, ayf, x_off0, x_off1, y_off0, y_off1, zf0, zf1,
                   cuep, vox1, vox2)
    return jnp.sum(out)
